# baseline (speedup 1.0000x reference)
import functools

import jax
import jax.numpy as jnp
from jax import lax
from jax.experimental import pallas as pl
from jax.experimental.pallas import tpu as pltpu
from jax.experimental.pallas import tpu_sc as plsc

MAX_LEN_ = 4096
D_MODEL_ = 1024
BATCH_ = 4
NC_ = 2
NS_ = 16
NW_ = NC_ * NS_
RPW_ = MAX_LEN_ // NW_
CROWS_ = 32
CH_ = CROWS_ * D_MODEL_
NCHUNK_ = RPW_ // CROWS_
LANES_ = 16


def _sc_body(x_hbm, t_hbm, o_hbm, xbuf, tbuf):
    wid = lax.axis_index("s") * NC_ + lax.axis_index("c")
    wbase = wid * (RPW_ * D_MODEL_)

    def add_vec(i, _):
        off = i * LANES_
        t = tbuf[pl.ds(off, LANES_)]
        plsc.addupdate(xbuf.at[pl.ds(off, LANES_)], t)
        return 0

    for k in range(NCHUNK_):
        t_off = wbase + k * CH_
        pltpu.sync_copy(t_hbm.at[pl.ds(t_off, CH_)], tbuf)
        for b in range(BATCH_):
            x_off = b * (MAX_LEN_ * D_MODEL_) + t_off
            pltpu.sync_copy(x_hbm.at[pl.ds(x_off, CH_)], xbuf)
            lax.fori_loop(0, CH_ // LANES_, add_vec, 0)
            pltpu.sync_copy(xbuf, o_hbm.at[pl.ds(x_off, CH_)])


_sc_add = functools.partial(
    pl.kernel,
    out_type=jax.ShapeDtypeStruct((BATCH_ * MAX_LEN_ * D_MODEL_,), jnp.float32),
    mesh=plsc.VectorSubcoreMesh(core_axis_name="c", subcore_axis_name="s"),
    scratch_types=[
        pltpu.VMEM((CH_,), jnp.float32),
        pltpu.VMEM((CH_,), jnp.float32),
    ],
)(_sc_body)


def kernel(x, pos_table):
    xf = x.reshape(-1)
    tf = pos_table.reshape(-1)
    out = _sc_add(xf, tf)
    return out.reshape(x.shape)


# speedup vs baseline: 1.6235x; 1.6235x over previous
import functools

import jax
import jax.numpy as jnp
from jax import lax
from jax.experimental import pallas as pl
from jax.experimental.pallas import tpu as pltpu
from jax.experimental.pallas import tpu_sc as plsc

MAX_LEN_ = 4096
D_MODEL_ = 1024
BATCH_ = 4
NC_ = 2
NS_ = 16
NW_ = NC_ * NS_
RPW_ = MAX_LEN_ // NW_
CROWS_ = 16
CH_ = CROWS_ * D_MODEL_
NCHUNK_ = RPW_ // CROWS_
NP_ = NCHUNK_ * BATCH_
LANES_ = 16
UNROLL_ = 8


def _sc_body(x_hbm, t_hbm, o_hbm,
             xb0, xb1, xb2, tb0, tb1,
             is0, is1, is2, os0, os1, os2, ts0, ts1):
    wid = lax.axis_index("s") * NC_ + lax.axis_index("c")
    wbase = wid * (RPW_ * D_MODEL_)
    xbufs = (xb0, xb1, xb2)
    tbufs = (tb0, tb1)
    isems = (is0, is1, is2)
    osems = (os0, os1, os2)
    tsems = (ts0, ts1)

    def x_off(p):
        k, b = p // BATCH_, p % BATCH_
        return b * (MAX_LEN_ * D_MODEL_) + wbase + k * CH_

    def start_in(p):
        return pltpu.async_copy(
            x_hbm.at[pl.ds(x_off(p), CH_)], xbufs[p % 3], isems[p % 3])

    def start_out(p):
        return pltpu.async_copy(
            xbufs[p % 3], o_hbm.at[pl.ds(x_off(p), CH_)], osems[p % 3])

    def start_tbl(k):
        return pltpu.async_copy(
            t_hbm.at[pl.ds(wbase + k * CH_, CH_)], tbufs[k % 2], tsems[k % 2])

    tbl_d = {0: start_tbl(0)}
    in_d = {0: start_in(0)}
    out_d = {}

    for p in range(NP_):
        k, b = p // BATCH_, p % BATCH_
        if p + 1 < NP_:
            if p - 2 >= 0:
                out_d[p - 2].wait()
            in_d[p + 1] = start_in(p + 1)
        if b == 0:
            if k + 1 < NCHUNK_:
                tbl_d[k + 1] = start_tbl(k + 1)
            tbl_d[k].wait()
        in_d[p].wait()
        xbuf = xbufs[p % 3]
        tbuf = tbufs[k % 2]

        def add_vec(i):
            off = i * LANES_
            plsc.addupdate(xbuf.at[pl.ds(off, LANES_)], tbuf[pl.ds(off, LANES_)])

        plsc.parallel_loop(0, CH_ // LANES_, 1, unroll=UNROLL_)(add_vec)
        out_d[p] = start_out(p)

    for p in range(NP_ - 3, NP_):
        out_d[p].wait()


_sc_add = functools.partial(
    pl.kernel,
    out_type=jax.ShapeDtypeStruct((BATCH_ * MAX_LEN_ * D_MODEL_,), jnp.float32),
    mesh=plsc.VectorSubcoreMesh(core_axis_name="c", subcore_axis_name="s"),
    scratch_types=(
        [pltpu.VMEM((CH_,), jnp.float32)] * 3
        + [pltpu.VMEM((CH_,), jnp.float32)] * 2
        + [pltpu.SemaphoreType.DMA] * 8
    ),
)(_sc_body)


def kernel(x, pos_table):
    xf = x.reshape(-1)
    tf = pos_table.reshape(-1)
    out = _sc_add(xf, tf)
    return out.reshape(x.shape)


# speedup vs baseline: 4.3875x; 2.7025x over previous
import functools

import jax
import jax.numpy as jnp
from jax import lax
from jax.experimental import pallas as pl
from jax.experimental.pallas import tpu as pltpu
from jax.experimental.pallas import tpu_sc as plsc

MAX_LEN_ = 4096
D_MODEL_ = 1024
BATCH_ = 4
NC_ = 2
NS_ = 16
NW_ = NC_ * NS_
RPW_ = MAX_LEN_ // NW_
CROWS_ = 16
NCHUNK_ = RPW_ // CROWS_
NP_ = NCHUNK_ * BATCH_
LANES_ = 16
VPR_ = D_MODEL_ // LANES_
UNROLL_ = 8


def _sc_body(x_hbm, t_hbm, o_hbm,
             xb0, xb1, xb2, tb0, tb1,
             is0, is1, is2, os0, os1, os2, ts0, ts1):
    wid = lax.axis_index("s") * NC_ + lax.axis_index("c")
    wrow = wid * RPW_
    xbufs = (xb0, xb1, xb2)
    tbufs = (tb0, tb1)
    isems = (is0, is1, is2)
    osems = (os0, os1, os2)
    tsems = (ts0, ts1)

    def x_row(p):
        k, b = p // BATCH_, p % BATCH_
        return b * MAX_LEN_ + wrow + k * CROWS_

    def start_in(p):
        return pltpu.async_copy(
            x_hbm.at[pl.ds(x_row(p), CROWS_), :], xbufs[p % 3], isems[p % 3])

    def start_out(p):
        return pltpu.async_copy(
            xbufs[p % 3], o_hbm.at[pl.ds(x_row(p), CROWS_), :], osems[p % 3])

    def start_tbl(k):
        return pltpu.async_copy(
            t_hbm.at[pl.ds(wrow + k * CROWS_, CROWS_), :],
            tbufs[k % 2], tsems[k % 2])

    tbl_d = {0: start_tbl(0)}
    in_d = {0: start_in(0)}
    out_d = {}

    for p in range(NP_):
        k, b = p // BATCH_, p % BATCH_
        if p + 1 < NP_:
            if p - 2 >= 0:
                out_d[p - 2].wait()
            in_d[p + 1] = start_in(p + 1)
        if b == 0:
            if k + 1 < NCHUNK_:
                tbl_d[k + 1] = start_tbl(k + 1)
            tbl_d[k].wait()
        in_d[p].wait()
        xbuf = xbufs[p % 3]
        tbuf = tbufs[k % 2]

        def add_vec(i):
            r = i // VPR_
            c = (i % VPR_) * LANES_
            plsc.addupdate(xbuf.at[r, pl.ds(c, LANES_)],
                           tbuf[r, pl.ds(c, LANES_)])

        plsc.parallel_loop(0, CROWS_ * VPR_, 1, unroll=UNROLL_)(add_vec)
        out_d[p] = start_out(p)

    for p in range(NP_ - 3, NP_):
        out_d[p].wait()


_sc_add = functools.partial(
    pl.kernel,
    out_type=jax.ShapeDtypeStruct((BATCH_ * MAX_LEN_, D_MODEL_), jnp.float32),
    mesh=plsc.VectorSubcoreMesh(core_axis_name="c", subcore_axis_name="s"),
    scratch_types=(
        [pltpu.VMEM((CROWS_, D_MODEL_), jnp.float32)] * 3
        + [pltpu.VMEM((CROWS_, D_MODEL_), jnp.float32)] * 2
        + [pltpu.SemaphoreType.DMA] * 8
    ),
)(_sc_body)


def kernel(x, pos_table):
    xf = x.reshape(BATCH_ * MAX_LEN_, D_MODEL_)
    out = _sc_add(xf, pos_table)
    return out.reshape(x.shape)


# speedup vs baseline: 6.2704x; 1.4291x over previous
import jax
import jax.numpy as jnp
from jax.experimental import pallas as pl

MAX_LEN_ = 4096
D_MODEL_ = 1024
BATCH_ = 4
BLOCK_ = 512


def _add_block(x_ref, t_ref, o_ref):
    o_ref[...] = x_ref[...] + t_ref[...]


def kernel(x, pos_table):
    b, L, d = x.shape
    xf = x.reshape(b * L, d)
    nt = L // BLOCK_
    out = pl.pallas_call(
        _add_block,
        out_shape=jax.ShapeDtypeStruct((b * L, d), x.dtype),
        grid=(nt, b),
        in_specs=[
            pl.BlockSpec((BLOCK_, d), lambda i, j: (j * nt + i, 0)),
            pl.BlockSpec((BLOCK_, d), lambda i, j: (i, 0)),
        ],
        out_specs=pl.BlockSpec((BLOCK_, d), lambda i, j: (j * nt + i, 0)),
    )(xf, pos_table)
    return out.reshape(b, L, d)


# speedup vs baseline: 6.9387x; 1.1066x over previous
import jax
import jax.numpy as jnp
from jax.experimental import pallas as pl

MAX_LEN_ = 4096
D_MODEL_ = 1024
BATCH_ = 4
BLOCK_ = 1024


def _add_block(x_ref, t_ref, o_ref):
    o_ref[...] = x_ref[...] + t_ref[...]


def kernel(x, pos_table):
    b, L, d = x.shape
    xf = x.reshape(b * L, d)
    nt = L // BLOCK_
    out = pl.pallas_call(
        _add_block,
        out_shape=jax.ShapeDtypeStruct((b * L, d), x.dtype),
        grid=(nt, b),
        in_specs=[
            pl.BlockSpec((BLOCK_, d), lambda i, j: (j * nt + i, 0)),
            pl.BlockSpec((BLOCK_, d), lambda i, j: (i, 0)),
        ],
        out_specs=pl.BlockSpec((BLOCK_, d), lambda i, j: (j * nt + i, 0)),
    )(xf, pos_table)
    return out.reshape(b, L, d)


# speedup vs baseline: 7.3184x; 1.0547x over previous
import jax
import jax.numpy as jnp
from jax.experimental import pallas as pl

MAX_LEN_ = 4096
D_MODEL_ = 1024
BATCH_ = 4
BLOCK_ = 2048


def _add_block(x_ref, t_ref, o_ref):
    o_ref[...] = x_ref[...] + t_ref[...]


def kernel(x, pos_table):
    b, L, d = x.shape
    xf = x.reshape(b * L, d)
    nt = L // BLOCK_
    out = pl.pallas_call(
        _add_block,
        out_shape=jax.ShapeDtypeStruct((b * L, d), x.dtype),
        grid=(nt, b),
        in_specs=[
            pl.BlockSpec((BLOCK_, d), lambda i, j: (j * nt + i, 0)),
            pl.BlockSpec((BLOCK_, d), lambda i, j: (i, 0)),
        ],
        out_specs=pl.BlockSpec((BLOCK_, d), lambda i, j: (j * nt + i, 0)),
    )(xf, pos_table)
    return out.reshape(b, L, d)
